# initial kernel scaffold (unmeasured)
import jax
import jax.numpy as jnp
from jax import lax
from jax.experimental import pallas as pl
from jax.experimental.pallas import tpu as pltpu

N_DEV = 32


def kernel(x, w_mat, scale_x, scale_w):
    k_total, k_shard = x.shape
    _, n = w_mat.shape
    m_per = k_total // N_DEV

    def body(x_ref, w_ref, sx_ref, sw_ref, out_ref, gather_ref,
             send_sems, recv_sems):
        me = lax.axis_index("i")

        barrier_sem = pltpu.get_barrier_semaphore()
        for d in range(1, N_DEV):
            peer = lax.rem(me + d, N_DEV)
            pl.semaphore_signal(
                barrier_sem, inc=1, device_id=(peer,),
                device_id_type=pl.DeviceIdType.MESH,
            )
        pl.semaphore_wait(barrier_sem, N_DEV - 1)

        gather_ref[:, pl.ds(me * k_shard, k_shard)] = (
            x_ref[pl.ds(me * m_per, m_per), :]
        )

        sends = []
        for d in range(1, N_DEV):
            dst = lax.rem(me + d, N_DEV)
            rdma = pltpu.make_async_remote_copy(
                src_ref=x_ref.at[pl.ds(dst * m_per, m_per), :],
                dst_ref=gather_ref.at[:, pl.ds(me * k_shard, k_shard)],
                send_sem=send_sems.at[d],
                recv_sem=recv_sems.at[me],
                device_id=(dst,),
                device_id_type=pl.DeviceIdType.MESH,
            )
            rdma.start()
            sends.append(rdma)

        for d in range(1, N_DEV):
            src = lax.rem(me + d, N_DEV)
            recv = pltpu.make_async_remote_copy(
                src_ref=x_ref.at[pl.ds(0, m_per), :],
                dst_ref=gather_ref.at[:, pl.ds(src * k_shard, k_shard)],
                send_sem=send_sems.at[0],
                recv_sem=recv_sems.at[src],
                device_id=(src,),
                device_id_type=pl.DeviceIdType.MESH,
            )
            recv.wait_recv()

        acc = lax.dot_general(
            gather_ref[:, :], w_ref[:, :],
            (((1,), (0,)), ((), ())),
            preferred_element_type=jnp.int32,
        )
        s = sx_ref[0] * sw_ref[0]
        y = acc.astype(jnp.float32) * s
        z = jnp.clip(y, -60.0, 60.0)
        out_ref[:, :] = y / (1.0 + jnp.exp(-z))

        for rdma in sends:
            rdma.wait_send()

    return pl.pallas_call(
        body,
        out_shape=jax.ShapeDtypeStruct((m_per, n), jnp.float32),
        in_specs=[
            pl.BlockSpec(memory_space=pltpu.VMEM),
            pl.BlockSpec(memory_space=pltpu.VMEM),
            pl.BlockSpec(memory_space=pltpu.SMEM),
            pl.BlockSpec(memory_space=pltpu.SMEM),
        ],
        out_specs=pl.BlockSpec(memory_space=pltpu.VMEM),
        scratch_shapes=[
            pltpu.VMEM((m_per, k_total), jnp.int8),
            pltpu.SemaphoreType.DMA((N_DEV,)),
            pltpu.SemaphoreType.DMA((N_DEV,)),
        ],
        compiler_params=pltpu.CompilerParams(collective_id=0),
    )(x, w_mat, scale_x, scale_w)


# baseline (device time: 44521 ns/iter reference)
import jax
import jax.numpy as jnp
from jax import lax
from jax.experimental import pallas as pl
from jax.experimental.pallas import tpu as pltpu

N_DEV = 32


def kernel(x, w_mat, scale_x, scale_w):
    k_total, k_shard = x.shape
    _, n = w_mat.shape
    m_per = k_total // N_DEV

    def body(x_ref, w_ref, sx_ref, sw_ref, out_ref, gather_ref,
             send_sems, recv_sems):
        me = lax.axis_index("i")

        barrier_sem = pltpu.get_barrier_semaphore()
        for d in range(1, N_DEV):
            peer = lax.rem(me + d, N_DEV)
            pl.semaphore_signal(
                barrier_sem, inc=1, device_id=(peer,),
                device_id_type=pl.DeviceIdType.MESH,
            )
        pl.semaphore_wait(barrier_sem, N_DEV - 1)

        gather_ref[:, pl.ds(me * k_shard, k_shard)] = (
            x_ref[pl.ds(me * m_per, m_per), :]
        )

        sends = []
        for d in range(1, N_DEV):
            dst = lax.rem(me + d, N_DEV)
            rdma = pltpu.make_async_remote_copy(
                src_ref=x_ref.at[pl.ds(dst * m_per, m_per), :],
                dst_ref=gather_ref.at[:, pl.ds(me * k_shard, k_shard)],
                send_sem=send_sems.at[d],
                recv_sem=recv_sems.at[me],
                device_id=(dst,),
                device_id_type=pl.DeviceIdType.MESH,
            )
            rdma.start()
            sends.append(rdma)

        for d in range(1, N_DEV):
            src = lax.rem(me + d, N_DEV)
            recv = pltpu.make_async_remote_copy(
                src_ref=x_ref.at[pl.ds(0, m_per), :],
                dst_ref=gather_ref.at[:, pl.ds(src * k_shard, k_shard)],
                send_sem=send_sems.at[0],
                recv_sem=recv_sems.at[src],
                device_id=(src,),
                device_id_type=pl.DeviceIdType.MESH,
            )
            recv.wait_recv()

        acc = lax.dot_general(
            gather_ref[:, :], w_ref[:, :],
            (((1,), (0,)), ((), ())),
            preferred_element_type=jnp.int32,
        )
        s = sx_ref[0] * sw_ref[0]
        y = acc.astype(jnp.float32) * s
        z = jnp.clip(y, -60.0, 60.0)
        out_ref[:, :] = y / (1.0 + jnp.exp(-z))

        for rdma in sends:
            rdma.wait_send()

    return pl.pallas_call(
        body,
        out_shape=jax.ShapeDtypeStruct((m_per, n), jnp.float32),
        in_specs=[
            pl.BlockSpec(memory_space=pltpu.VMEM),
            pl.BlockSpec(memory_space=pltpu.VMEM),
            pl.BlockSpec(memory_space=pltpu.SMEM),
            pl.BlockSpec(memory_space=pltpu.SMEM),
        ],
        out_specs=pl.BlockSpec(memory_space=pltpu.VMEM),
        scratch_shapes=[
            pltpu.VMEM((m_per, k_total), jnp.int8),
            pltpu.SemaphoreType.DMA((N_DEV,)),
            pltpu.SemaphoreType.DMA((N_DEV,)),
        ],
        compiler_params=pltpu.CompilerParams(
            collective_id=0,
            vmem_limit_bytes=100 * 1024 * 1024,
        ),
    )(x, w_mat, scale_x, scale_w)


# device time: 35565 ns/iter; 1.2518x vs baseline; 1.2518x over previous
import jax
import jax.numpy as jnp
from jax import lax
from jax.experimental import pallas as pl
from jax.experimental.pallas import tpu as pltpu

N_DEV = 32
KCH = 512


def kernel(x, w_mat, scale_x, scale_w):
    k_total, k_shard = x.shape
    _, n = w_mat.shape
    m_per = k_total // N_DEV
    nch = k_total // KCH
    blk_per_ch = KCH // k_shard

    def body(x_ref, w_ref, sx_ref, sw_ref, out_ref, gather_ref, wbuf_ref,
             send_sems, recv_sems, wdma_sems):
        me = lax.axis_index("i")

        wcopies = []
        for c in range(nch):
            cp = pltpu.make_async_copy(
                w_ref.at[pl.ds(c * KCH, KCH), :],
                wbuf_ref.at[c],
                wdma_sems.at[c],
            )
            cp.start()
            wcopies.append(cp)

        barrier_sem = pltpu.get_barrier_semaphore()
        for d in range(1, N_DEV):
            peer = lax.rem(me + d, N_DEV)
            pl.semaphore_signal(
                barrier_sem, inc=1, device_id=(peer,),
                device_id_type=pl.DeviceIdType.MESH,
            )
        pl.semaphore_wait(barrier_sem, N_DEV - 1)

        sends = []
        for d in range(N_DEV):
            dst = lax.rem(me + d, N_DEV)
            rdma = pltpu.make_async_remote_copy(
                src_ref=x_ref.at[pl.ds(dst * m_per, m_per), :],
                dst_ref=gather_ref.at[:, pl.ds(me * k_shard, k_shard)],
                send_sem=send_sems.at[d],
                recv_sem=recv_sems.at[me],
                device_id=(dst,),
                device_id_type=pl.DeviceIdType.MESH,
            )
            rdma.start()
            sends.append(rdma)

        def wait_block(j):
            recv = pltpu.make_async_remote_copy(
                src_ref=x_ref.at[pl.ds(0, m_per), :],
                dst_ref=gather_ref.at[:, pl.ds(j * k_shard, k_shard)],
                send_sem=send_sems.at[0],
                recv_sem=recv_sems.at[j],
                device_id=(j,),
                device_id_type=pl.DeviceIdType.MESH,
            )
            recv.wait_recv()

        acc = None
        for c in range(nch):
            wcopies[c].wait()
            for b in range(blk_per_ch):
                wait_block(c * blk_per_ch + b)
            part = lax.dot_general(
                gather_ref[:, pl.ds(c * KCH, KCH)], wbuf_ref[c],
                (((1,), (0,)), ((), ())),
                preferred_element_type=jnp.int32,
            )
            acc = part if acc is None else acc + part

        s = sx_ref[0] * sw_ref[0]
        y = acc.astype(jnp.float32) * s
        z = jnp.clip(y, -60.0, 60.0)
        out_ref[:, :] = y / (1.0 + jnp.exp(-z))

        for rdma in sends:
            rdma.wait_send()

    return pl.pallas_call(
        body,
        out_shape=jax.ShapeDtypeStruct((m_per, n), jnp.float32),
        in_specs=[
            pl.BlockSpec(memory_space=pltpu.VMEM),
            pl.BlockSpec(memory_space=pl.ANY),
            pl.BlockSpec(memory_space=pltpu.SMEM),
            pl.BlockSpec(memory_space=pltpu.SMEM),
        ],
        out_specs=pl.BlockSpec(memory_space=pltpu.VMEM),
        scratch_shapes=[
            pltpu.VMEM((m_per, k_total), jnp.int8),
            pltpu.VMEM((nch, KCH, n), jnp.int8),
            pltpu.SemaphoreType.DMA((N_DEV,)),
            pltpu.SemaphoreType.DMA((N_DEV,)),
            pltpu.SemaphoreType.DMA((nch,)),
        ],
        compiler_params=pltpu.CompilerParams(
            collective_id=0,
            vmem_limit_bytes=100 * 1024 * 1024,
        ),
    )(x, w_mat, scale_x, scale_w)


# device time: 33832 ns/iter; 1.3159x vs baseline; 1.0512x over previous
import jax
import jax.numpy as jnp
from jax import lax
from jax.experimental import pallas as pl
from jax.experimental.pallas import tpu as pltpu

N_DEV = 32
KCH = 512


def kernel(x, w_mat, scale_x, scale_w):
    k_total, k_shard = x.shape
    _, n = w_mat.shape
    m_per = k_total // N_DEV
    nch = k_total // KCH
    blk_per_ch = KCH // k_shard

    def body(x_ref, w_ref, sx_ref, sw_ref, out_ref, gather_ref, wbuf_ref,
             send_sems, recv_sems, wdma_sems):
        me = lax.axis_index("i")
        my_ch = lax.div(me, blk_per_ch)

        wcopies = []
        for t in range(nch):
            c = lax.rem(my_ch - 1 - t + 2 * nch, nch)
            cp = pltpu.make_async_copy(
                w_ref.at[pl.ds(c * KCH, KCH), :],
                wbuf_ref.at[t],
                wdma_sems.at[t],
            )
            cp.start()
            wcopies.append((c, cp))

        barrier_sem = pltpu.get_barrier_semaphore()
        for d in range(1, N_DEV):
            peer = lax.rem(me + d, N_DEV)
            pl.semaphore_signal(
                barrier_sem, inc=1, device_id=(peer,),
                device_id_type=pl.DeviceIdType.MESH,
            )
        pl.semaphore_wait(barrier_sem, N_DEV - 1)

        sends = []
        for d in range(N_DEV):
            dst = lax.rem(me + d, N_DEV)
            rdma = pltpu.make_async_remote_copy(
                src_ref=x_ref.at[pl.ds(dst * m_per, m_per), :],
                dst_ref=gather_ref.at[:, pl.ds(me * k_shard, k_shard)],
                send_sem=send_sems.at[d],
                recv_sem=recv_sems.at[me],
                device_id=(dst,),
                device_id_type=pl.DeviceIdType.MESH,
            )
            rdma.start()
            sends.append(rdma)

        def wait_block(j):
            recv = pltpu.make_async_remote_copy(
                src_ref=x_ref.at[pl.ds(0, m_per), :],
                dst_ref=gather_ref.at[:, pl.ds(j * k_shard, k_shard)],
                send_sem=send_sems.at[0],
                recv_sem=recv_sems.at[j],
                device_id=(j,),
                device_id_type=pl.DeviceIdType.MESH,
            )
            recv.wait_recv()

        acc = None
        for t in range(nch):
            c, cp = wcopies[t]
            cp.wait()
            for b in range(blk_per_ch):
                wait_block(c * blk_per_ch + (blk_per_ch - 1 - b))
            part = lax.dot_general(
                gather_ref[:, pl.ds(c * KCH, KCH)], wbuf_ref[t],
                (((1,), (0,)), ((), ())),
                preferred_element_type=jnp.int32,
            )
            acc = part if acc is None else acc + part

        s = sx_ref[0] * sw_ref[0]
        y = acc.astype(jnp.float32) * s
        z = jnp.clip(y, -60.0, 60.0)
        out_ref[:, :] = y / (1.0 + jnp.exp(-z))

        for rdma in sends:
            rdma.wait_send()

    return pl.pallas_call(
        body,
        out_shape=jax.ShapeDtypeStruct((m_per, n), jnp.float32),
        in_specs=[
            pl.BlockSpec(memory_space=pltpu.VMEM),
            pl.BlockSpec(memory_space=pl.ANY),
            pl.BlockSpec(memory_space=pltpu.SMEM),
            pl.BlockSpec(memory_space=pltpu.SMEM),
        ],
        out_specs=pl.BlockSpec(memory_space=pltpu.VMEM),
        scratch_shapes=[
            pltpu.VMEM((m_per, k_total), jnp.int8),
            pltpu.VMEM((nch, KCH, n), jnp.int8),
            pltpu.SemaphoreType.DMA((N_DEV,)),
            pltpu.SemaphoreType.DMA((N_DEV,)),
            pltpu.SemaphoreType.DMA((nch,)),
        ],
        compiler_params=pltpu.CompilerParams(
            collective_id=0,
            vmem_limit_bytes=100 * 1024 * 1024,
        ),
    )(x, w_mat, scale_x, scale_w)
